# baseline (speedup 1.0000x reference)
import functools

import jax
import jax.numpy as jnp
from jax import lax
from jax.experimental import pallas as pl
from jax.experimental.pallas import tpu as pltpu

C = 8
F = 128
B = 4
K = 512
V = 65536
THETA = 0.9
TAU = 0.1

VB = 4096


def _tc1_body(proba_ref, y_ref, emb_ref, sums_ref, counts_ref, w_ref, sel_ref):
    b = pl.program_id(0)
    v = pl.program_id(1)

    yf = (y_ref[0] > 0).astype(jnp.float32)
    emb = emb_ref[0]

    @pl.when(jnp.logical_and(b == 0, v == 0))
    def _init():
        sums_ref[...] = jnp.zeros_like(sums_ref)
        counts_ref[...] = jnp.zeros_like(counts_ref)

    sums_ref[...] += lax.dot_general(
        yf, emb, (((1,), (1,)), ((), ())), preferred_element_type=jnp.float32)
    counts_ref[...] += jnp.sum(yf, axis=1, keepdims=True)

    p = proba_ref[0]
    w = p[0]
    for c in range(1, C):
        w = w * p[c]
    w_ref[0, 0] = w

    y0 = yf[0]
    ysum = jnp.sum(yf, axis=0)
    sel_ref[0, 0] = jnp.where(jnp.logical_or(y0 > 0.0, ysum == 0.0), 1.0, 0.0)


def _tc1(proba, y, emb):
    grid = (B, V // VB)
    return pl.pallas_call(
        _tc1_body,
        grid=grid,
        in_specs=[
            pl.BlockSpec((1, C, VB), lambda b, v: (b, 0, v)),
            pl.BlockSpec((1, C, VB), lambda b, v: (b, 0, v)),
            pl.BlockSpec((1, F, VB), lambda b, v: (b, 0, v)),
        ],
        out_specs=[
            pl.BlockSpec((C, F), lambda b, v: (0, 0)),
            pl.BlockSpec((C, 1), lambda b, v: (0, 0)),
            pl.BlockSpec((1, 1, VB), lambda b, v: (b * (V // VB) + v, 0, 0)),
            pl.BlockSpec((1, 1, VB), lambda b, v: (b * (V // VB) + v, 0, 0)),
        ],
        out_shape=[
            jax.ShapeDtypeStruct((C, F), jnp.float32),
            jax.ShapeDtypeStruct((C, 1), jnp.float32),
            jax.ShapeDtypeStruct((B * V // VB, 1, VB), jnp.float32),
            jax.ShapeDtypeStruct((B * V // VB, 1, VB), jnp.float32),
        ],
    )(proba, y, emb)


def _tc3_body(he_ref, selk_ref, sums_ref, counts_ref, avg0_ref, out_ref):
    counts = counts_ref[...]
    sums = sums_ref[...]
    avg0 = avg0_ref[0]
    means = sums / jnp.maximum(counts, 1.0)
    avg = jnp.where(counts > 0.0, avg0 * (1.0 - THETA) + means * THETA, avg0)

    he = he_ref[...]
    acc = jnp.zeros_like(he)
    for c in range(1, C):
        acc += jnp.exp(he * (avg[c][None, :] / TAU))
    rows = jnp.sum(jnp.log(acc), axis=1, keepdims=True)
    rows = rows * selk_ref[...][:, :1]
    seg = (lax.broadcasted_iota(jnp.int32, (B * K, B), 0) // K
           == lax.broadcasted_iota(jnp.int32, (B * K, B), 1)).astype(jnp.float32)
    out_ref[...] = -lax.dot_general(
        rows, seg, (((0,), (0,)), ((), ())),
        preferred_element_type=jnp.float32)


def _tc3(he, selk, sums, counts, avg0):
    return pl.pallas_call(
        _tc3_body,
        out_shape=jax.ShapeDtypeStruct((1, B), jnp.float32),
    )(he, selk, sums, counts, avg0).reshape(B)


def _topk_gather_placeholder(weights, sel0, emb):
    _, idx = lax.top_k(weights, K)
    he = jnp.take_along_axis(
        emb.reshape(B, F, V), idx[:, None, :], axis=2)
    he = jnp.transpose(he, (0, 2, 1)).reshape(B * K, F)
    selk = jnp.take_along_axis(sel0, idx, axis=1).reshape(B * K, 1)
    selk = jnp.broadcast_to(selk, (B * K, 16))
    return he, selk


def kernel(proba, y, embeddings, average_representations):
    proba = proba.reshape(B, C, V)
    y = y.reshape(B, C, V)
    embeddings = embeddings.reshape(B, F, V)
    sums, counts, weights, sel0 = _tc1(proba, y, embeddings)
    weights = weights.reshape(B, V)
    sel0 = sel0.reshape(B, V)
    he, selk = _topk_gather_placeholder(weights, sel0, embeddings)
    return _tc3(he, selk, sums, counts, average_representations)


# speedup vs baseline: 1.9698x; 1.9698x over previous
import functools

import jax
import jax.numpy as jnp
from jax import lax
from jax.experimental import pallas as pl
from jax.experimental.pallas import tpu as pltpu
from jax.experimental.pallas import tpu_sc as plsc

C = 8
F = 128
B = 4
K = 512
V = 65536
THETA = 0.9
TAU = 0.1

VB = 4096


def _tc1_body(proba_ref, y_ref, emb_ref, sums_ref, counts_ref, w_ref, sel_ref):
    b = pl.program_id(0)
    v = pl.program_id(1)

    yf = (y_ref[0] > 0).astype(jnp.float32)
    emb = emb_ref[0]

    @pl.when(jnp.logical_and(b == 0, v == 0))
    def _init():
        sums_ref[...] = jnp.zeros_like(sums_ref)
        counts_ref[...] = jnp.zeros_like(counts_ref)

    sums_ref[...] += lax.dot_general(
        yf, emb, (((1,), (1,)), ((), ())), preferred_element_type=jnp.float32)
    counts_ref[...] += jnp.sum(yf, axis=1, keepdims=True)

    p = proba_ref[0]
    w = p[0]
    for c in range(1, C):
        w = w * p[c]
    w_ref[0, 0] = w

    y0 = yf[0]
    ysum = jnp.sum(yf, axis=0)
    sel_ref[0, 0] = jnp.where(jnp.logical_or(y0 > 0.0, ysum == 0.0), 1.0, 0.0)


def _tc1(proba, y, emb):
    grid = (B, V // VB)
    return pl.pallas_call(
        _tc1_body,
        grid=grid,
        in_specs=[
            pl.BlockSpec((1, C, VB), lambda b, v: (b, 0, v)),
            pl.BlockSpec((1, C, VB), lambda b, v: (b, 0, v)),
            pl.BlockSpec((1, F, VB), lambda b, v: (b, 0, v)),
        ],
        out_specs=[
            pl.BlockSpec((C, F), lambda b, v: (0, 0)),
            pl.BlockSpec((C, 1), lambda b, v: (0, 0)),
            pl.BlockSpec((1, 1, VB), lambda b, v: (b * (V // VB) + v, 0, 0)),
            pl.BlockSpec((1, 1, VB), lambda b, v: (b * (V // VB) + v, 0, 0)),
        ],
        out_shape=[
            jax.ShapeDtypeStruct((C, F), jnp.float32),
            jax.ShapeDtypeStruct((C, 1), jnp.float32),
            jax.ShapeDtypeStruct((B * V // VB, 1, VB), jnp.float32),
            jax.ShapeDtypeStruct((B * V // VB, 1, VB), jnp.float32),
        ],
    )(proba, y, emb)


def _tc3_body(he_ref, selk_ref, sums_ref, counts_ref, avg0_ref, out_ref):
    counts = counts_ref[...]
    sums = sums_ref[...]
    avg0 = avg0_ref[0]
    means = sums / jnp.maximum(counts, 1.0)
    avg = jnp.where(counts > 0.0, avg0 * (1.0 - THETA) + means * THETA, avg0)

    he = he_ref[...]
    acc = jnp.zeros_like(he)
    for c in range(1, C):
        acc += jnp.exp(he * (avg[c][None, :] / TAU))
    rows = jnp.sum(jnp.log(acc), axis=1, keepdims=True)
    rows = rows * selk_ref[...][:, :1]
    seg = (lax.broadcasted_iota(jnp.int32, (B * K, B), 0) // K
           == lax.broadcasted_iota(jnp.int32, (B * K, B), 1)).astype(jnp.float32)
    out_ref[...] = -lax.dot_general(
        rows, seg, (((0,), (0,)), ((), ())),
        preferred_element_type=jnp.float32)


def _tc3(he, selk, sums, counts, avg0):
    return pl.pallas_call(
        _tc3_body,
        out_shape=jax.ShapeDtypeStruct((1, B), jnp.float32),
    )(he, selk, sums, counts, avg0).reshape(B)


NT = 16
NCORES = 2
TPT = V // NT
NVR = TPT // 16
PADROWS = NCORES * NT * 16


def _sc2_body(w_hbm, s_hbm, e_hbm, he_hbm, selk_hbm,
              w_v, hist_v, tot_v, gtot_v, ctmp_v, exch_v, cnt_v, idxsel_v,
              gidx_v, gdst_v, ht_v, sidx_v, sdst_v, sexp_v, crow_v,
              sh_hist, sh_cnt, sem_g, sem_s, sem_r):
    cid = lax.axis_index("c")
    tid = lax.axis_index("s")
    lanes = lax.iota(jnp.int32, 16)
    ones16 = jnp.ones((16,), jnp.int32)
    zeros16 = jnp.zeros((16,), jnp.int32)

    for bi in range(B // NCORES):
        b = cid + NCORES * bi
        base = b * V + tid * TPT
        pltpu.sync_copy(w_hbm.at[pl.ds(base, TPT)], w_v)

        prefix = jnp.int32(0)
        need = jnp.int32(K)
        for level in range(4):
            shift = 24 - 8 * level

            def zbody(i, _):
                hist_v[pl.ds(i * 16, 16)] = zeros16
                return 0
            lax.fori_loop(0, NVR, zbody, 0)

            def hbody(i, _):
                k = lax.bitcast_convert_type(w_v[pl.ds(i * 16, 16)], jnp.int32)
                byte = lax.shift_right_logical(k, shift) & 0xFF
                slot = byte * 16 + lanes
                if level == 0:
                    plsc.addupdate_scatter(hist_v, [slot], ones16)
                else:
                    cand = lax.shift_right_logical(k, shift + 8) == \
                        lax.shift_right_logical(prefix, shift + 8)
                    plsc.addupdate_scatter(hist_v, [slot], ones16, mask=cand)
                return 0
            lax.fori_loop(0, NVR, hbody, 0)

            def tbody(g, _):
                for r in range(16):
                    row = hist_v[pl.ds((g * 16 + r) * 16, 16)]
                    ctmp_v[pl.ds(r * 16, 16)] = plsc.cumsum(row)
                tot_v[pl.ds(g * 16, 16)] = plsc.load_gather(
                    ctmp_v, [lanes * 16 + 15])
                return 0
            lax.fori_loop(0, 16, tbody, 0)

            pltpu.sync_copy(tot_v, sh_hist.at[tid])
            plsc.subcore_barrier()
            pltpu.sync_copy(sh_hist, exch_v)

            def gbody(g, _):
                acc = zeros16
                for t in range(NT):
                    acc = acc + exch_v[t, pl.ds(g * 16, 16)]
                gtot_v[pl.ds(g * 16, 16)] = acc
                return 0
            lax.fori_loop(0, 16, gbody, 0)
            plsc.subcore_barrier()

            def wbody(g, carry):
                cum, bstar, ngt_above = carry
                gbase = 256 - (g + 1) * 16
                row = gtot_v[pl.ds(gbase, 16)]
                rv = lax.rev(row, (0,))
                cs = plsc.cumsum(rv) + cum
                newcum = jnp.max(cs)
                found = jnp.logical_and(newcum >= need, cum < need)
                j = jnp.max(plsc.all_reduce_ffs(cs >= need))
                excl = cs - rv
                prevj = jnp.max(jnp.where(lanes == j, excl, 0))
                bstar = jnp.where(found, gbase + 15 - j, bstar)
                ngt_above = jnp.where(found, prevj, ngt_above)
                return (newcum, bstar, ngt_above)
            _, bstar, ngt_above = lax.fori_loop(
                0, 16, wbody, (jnp.int32(0), jnp.int32(0), jnp.int32(0)))
            need = need - ngt_above
            prefix = prefix | lax.shift_left(bstar, shift)

        tstar = prefix

        def cbody(i, carry):
            ngt_vec, neq_vec = carry
            k = lax.bitcast_convert_type(w_v[pl.ds(i * 16, 16)], jnp.int32)
            ngt_vec = ngt_vec + plsc.all_reduce_population_count(k > tstar)
            neq_vec = neq_vec + plsc.all_reduce_population_count(k == tstar)
            return (ngt_vec, neq_vec)
        ngt_vec, neq_vec = lax.fori_loop(0, NVR, cbody, (zeros16, zeros16))
        ngt = jnp.max(ngt_vec)
        neq = jnp.max(neq_vec)

        crow_v[...] = jnp.where(lanes == 0, ngt, jnp.where(lanes == 1, neq, 0))
        pltpu.sync_copy(crow_v, sh_cnt.at[pl.ds(tid * 16, 16)])
        plsc.subcore_barrier()
        pltpu.sync_copy(sh_cnt, cnt_v)

        gcol = plsc.load_gather(cnt_v, [lanes * 16])
        ecol = plsc.load_gather(cnt_v, [lanes * 16 + 1])
        gt_all = jnp.sum(gcol)
        quota_eq = jnp.int32(K) - gt_all
        ec_excl = plsc.cumsum(ecol) - ecol
        take_vec = jnp.clip(quota_eq - ec_excl, 0, ecol)
        before = lanes < tid
        woff = jnp.sum(jnp.where(before, gcol + take_vec, 0))
        my_take = jnp.max(jnp.where(lanes == tid, take_vec, 0))
        my_nsel = ngt + my_take
        plsc.subcore_barrier()

        def pbody(i, carry):
            off, eqseen = carry
            k = lax.bitcast_convert_type(w_v[pl.ds(i * 16, 16)], jnp.int32)
            gt = k > tstar
            eq = k == tstar
            eqrank = plsc.cumsum(eq.astype(jnp.int32)) + eqseen
            sel = jnp.logical_or(gt, jnp.logical_and(eq, eqrank <= my_take))
            gv = tid * TPT + i * 16 + lanes
            plsc.store_compressed(idxsel_v.at[pl.ds(off, 16)], gv, mask=sel)
            off = off + jnp.max(plsc.all_reduce_population_count(sel))
            eqseen = eqseen + jnp.max(plsc.all_reduce_population_count(eq))
            return (off, eqseen)
        off, _ = lax.fori_loop(0, NVR, pbody, (jnp.int32(0), jnp.int32(0)))
        idxsel_v[pl.ds(off, 16)] = jnp.full((16,), tid * TPT, jnp.int32)

        nchunks = (my_nsel + 15) // 16
        dump_base = B * K + (cid * NT + tid) * 16

        def chunk_body(ch, _):
            vidx = idxsel_v[pl.ds(ch * 16, 16)]

            def fbody(f, _):
                r = f // 8
                s = f % 8
                gidx_v[r, pl.ds(s * 16, 16)] = vidx + (b * F + f) * V
                return 0
            lax.fori_loop(0, F, fbody, 0)
            cps_g = []
            for r in range(16):
                cps_g.append(pltpu.async_copy(
                    e_hbm.at[gidx_v.at[r]],
                    gdst_v.at[pl.ds(r * 128, 128)], sem_g))
            sidx_v[...] = vidx + b * V
            cp_s = pltpu.async_copy(s_hbm.at[sidx_v], sdst_v, sem_s)
            for cp in cps_g:
                cp.wait()
            cp_s.wait()

            for j in range(16):
                for fg in range(F // 16):
                    ht_v[pl.ds(j * F + fg * 16, 16)] = plsc.load_gather(
                        gdst_v, [(fg * 16 + lanes) * 16 + j])
            svec = sdst_v[...]
            for cc in range(16):
                plsc.store_scatter(sexp_v, [lanes * 16 + cc], svec)

            nrem = my_nsel - ch * 16
            rowbase = b * K + woff + ch * 16
            cps = []
            for j in range(16):
                row = jnp.where(j < nrem, rowbase + j, dump_base + j)
                cps.append(pltpu.async_copy(
                    ht_v.at[pl.ds(j * F, F)],
                    he_hbm.at[pl.ds(row * F, F)], sem_r))
                cps.append(pltpu.async_copy(
                    sexp_v.at[pl.ds(j * 16, 16)],
                    selk_hbm.at[pl.ds(row * 16, 16)], sem_r))
            for cp in cps:
                cp.wait()
            return 0
        lax.fori_loop(0, nchunks, chunk_body, 0)
        plsc.subcore_barrier()


def _sc2(weights, sel0, emb_flat):
    rows = B * K + PADROWS
    mesh = plsc.VectorSubcoreMesh(core_axis_name="c", subcore_axis_name="s", num_cores=NCORES, num_subcores=NT)
    f = pl.kernel(
        _sc2_body,
        out_type=[
            jax.ShapeDtypeStruct((rows * F,), jnp.float32),
            jax.ShapeDtypeStruct((rows * 16,), jnp.float32),
        ],
        mesh=mesh,
        compiler_params=pltpu.CompilerParams(needs_layout_passes=False),
        scratch_types=[
            pltpu.VMEM((TPT,), jnp.float32),
            pltpu.VMEM((4096,), jnp.int32),
            pltpu.VMEM((256,), jnp.int32),
            pltpu.VMEM((256,), jnp.int32),
            pltpu.VMEM((256,), jnp.int32),
            pltpu.VMEM((NT, 256), jnp.int32),
            pltpu.VMEM((NT * 16,), jnp.int32),
            pltpu.VMEM((K + 32,), jnp.int32),
            pltpu.VMEM((16, 128), jnp.int32),
            pltpu.VMEM((16 * F,), jnp.float32),
            pltpu.VMEM((16 * F,), jnp.float32),
            pltpu.VMEM((16,), jnp.int32),
            pltpu.VMEM((16,), jnp.float32),
            pltpu.VMEM((256,), jnp.float32),
            pltpu.VMEM((16,), jnp.int32),
            pltpu.VMEM_SHARED((NT, 256), jnp.int32),
            pltpu.VMEM_SHARED((NT * 16,), jnp.int32),
            pltpu.SemaphoreType.DMA,
            pltpu.SemaphoreType.DMA,
            pltpu.SemaphoreType.DMA,
        ],
    )
    he_flat, selk_flat = f(weights.reshape(-1), sel0.reshape(-1), emb_flat)
    he = he_flat.reshape(rows, F)[:B * K]
    selk = selk_flat.reshape(rows, 16)[:B * K]
    return he, selk


def _topk_gather_placeholder(weights, sel0, emb):
    _, idx = lax.top_k(weights, K)
    he = jnp.take_along_axis(
        emb.reshape(B, F, V), idx[:, None, :], axis=2)
    he = jnp.transpose(he, (0, 2, 1)).reshape(B * K, F)
    selk = jnp.take_along_axis(sel0, idx, axis=1).reshape(B * K, 1)
    selk = jnp.broadcast_to(selk, (B * K, 16))
    return he, selk


def kernel(proba, y, embeddings, average_representations):
    proba = proba.reshape(B, C, V)
    y = y.reshape(B, C, V)
    embeddings = embeddings.reshape(B, F, V)
    sums, counts, weights, sel0 = _tc1(proba, y, embeddings)
    he, selk = _sc2(weights, sel0, embeddings.reshape(-1))
    return _tc3(he, selk, sums, counts, average_representations)


# speedup vs baseline: 2.3262x; 1.1809x over previous
import functools

import jax
import jax.numpy as jnp
from jax import lax
from jax.experimental import pallas as pl
from jax.experimental.pallas import tpu as pltpu
from jax.experimental.pallas import tpu_sc as plsc

C = 8
F = 128
B = 4
K = 512
V = 65536
THETA = 0.9
TAU = 0.1

VB = 4096


def _tc1_body(proba_ref, y_ref, emb_ref, sums_ref, counts_ref, w_ref, sel_ref, et_ref):
    b = pl.program_id(0)
    v = pl.program_id(1)

    yf = (y_ref[0] > 0).astype(jnp.float32)
    emb = emb_ref[0]

    @pl.when(jnp.logical_and(b == 0, v == 0))
    def _init():
        sums_ref[...] = jnp.zeros_like(sums_ref)
        counts_ref[...] = jnp.zeros_like(counts_ref)

    sums_ref[...] += lax.dot_general(
        yf, emb, (((1,), (1,)), ((), ())), preferred_element_type=jnp.float32)
    counts_ref[...] += jnp.sum(yf, axis=1, keepdims=True)

    p = proba_ref[0]
    w = p[0]
    for c in range(1, C):
        w = w * p[c]
    w_ref[0, 0] = w

    y0 = yf[0]
    ysum = jnp.sum(yf, axis=0)
    sel_ref[0, 0] = jnp.where(jnp.logical_or(y0 > 0.0, ysum == 0.0), 1.0, 0.0)

    et_ref[0] = emb.T


def _tc1(proba, y, emb):
    grid = (B, V // VB)
    return pl.pallas_call(
        _tc1_body,
        grid=grid,
        in_specs=[
            pl.BlockSpec((1, C, VB), lambda b, v: (b, 0, v)),
            pl.BlockSpec((1, C, VB), lambda b, v: (b, 0, v)),
            pl.BlockSpec((1, F, VB), lambda b, v: (b, 0, v)),
        ],
        out_specs=[
            pl.BlockSpec((C, F), lambda b, v: (0, 0)),
            pl.BlockSpec((C, 1), lambda b, v: (0, 0)),
            pl.BlockSpec((1, 1, VB), lambda b, v: (b * (V // VB) + v, 0, 0)),
            pl.BlockSpec((1, 1, VB), lambda b, v: (b * (V // VB) + v, 0, 0)),
            pl.BlockSpec((1, VB, F), lambda b, v: (b, v, 0)),
        ],
        out_shape=[
            jax.ShapeDtypeStruct((C, F), jnp.float32),
            jax.ShapeDtypeStruct((C, 1), jnp.float32),
            jax.ShapeDtypeStruct((B * V // VB, 1, VB), jnp.float32),
            jax.ShapeDtypeStruct((B * V // VB, 1, VB), jnp.float32),
            jax.ShapeDtypeStruct((B, V, F), jnp.float32),
        ],
    )(proba, y, emb)


def _tc3_body(he_ref, selk_ref, sums_ref, counts_ref, avg0_ref, out_ref):
    counts = counts_ref[...]
    sums = sums_ref[...]
    avg0 = avg0_ref[0]
    means = sums / jnp.maximum(counts, 1.0)
    avg = jnp.where(counts > 0.0, avg0 * (1.0 - THETA) + means * THETA, avg0)

    he = he_ref[...]
    acc = jnp.zeros_like(he)
    for c in range(1, C):
        acc += jnp.exp(he * (avg[c][None, :] / TAU))
    rows = jnp.sum(jnp.log(acc), axis=1, keepdims=True)
    rows = rows * selk_ref[...][:, :1]
    seg = (lax.broadcasted_iota(jnp.int32, (B * K, B), 0) // K
           == lax.broadcasted_iota(jnp.int32, (B * K, B), 1)).astype(jnp.float32)
    out_ref[...] = -lax.dot_general(
        rows, seg, (((0,), (0,)), ((), ())),
        preferred_element_type=jnp.float32)


def _tc3(he, selk, sums, counts, avg0):
    return pl.pallas_call(
        _tc3_body,
        out_shape=jax.ShapeDtypeStruct((1, B), jnp.float32),
    )(he, selk, sums, counts, avg0).reshape(B)


NT = 16
NCORES = 2
TPT = V // NT
NVR = TPT // 16
PADROWS = NCORES * NT * 16


def _sc2_body(w_hbm, s_hbm, e_hbm, he_hbm, selk_hbm,
              w_v, hist_v, tot_v, gtot_v, ctmp_v, exch_v, cnt_v, idxsel_v,
              grow_v, sidx_v, sdst_v, sexp_v, crow_v,
              sh_hist, sh_cnt, sem_g, sem_s, sem_r):
    cid = lax.axis_index("c")
    tid = lax.axis_index("s")
    lanes = lax.iota(jnp.int32, 16)
    ones16 = jnp.ones((16,), jnp.int32)
    zeros16 = jnp.zeros((16,), jnp.int32)

    for bi in range(B // NCORES):
        b = cid + NCORES * bi
        base = b * V + tid * TPT
        pltpu.sync_copy(w_hbm.at[pl.ds(base, TPT)], w_v)

        prefix = jnp.int32(0)
        need = jnp.int32(K)
        for level in range(4):
            shift = 24 - 8 * level

            def zbody(i, _):
                hist_v[pl.ds(i * 16, 16)] = zeros16
                return 0
            lax.fori_loop(0, NVR, zbody, 0)

            def hbody(i, _):
                k = lax.bitcast_convert_type(w_v[pl.ds(i * 16, 16)], jnp.int32)
                byte = lax.shift_right_logical(k, shift) & 0xFF
                slot = byte * 16 + lanes
                if level == 0:
                    plsc.addupdate_scatter(hist_v, [slot], ones16)
                else:
                    cand = lax.shift_right_logical(k, shift + 8) == \
                        lax.shift_right_logical(prefix, shift + 8)
                    plsc.addupdate_scatter(hist_v, [slot], ones16, mask=cand)
                return 0
            lax.fori_loop(0, NVR, hbody, 0)

            def tbody(g, _):
                for r in range(16):
                    row = hist_v[pl.ds((g * 16 + r) * 16, 16)]
                    ctmp_v[pl.ds(r * 16, 16)] = plsc.cumsum(row)
                tot_v[pl.ds(g * 16, 16)] = plsc.load_gather(
                    ctmp_v, [lanes * 16 + 15])
                return 0
            lax.fori_loop(0, 16, tbody, 0)

            pltpu.sync_copy(tot_v, sh_hist.at[tid])
            plsc.subcore_barrier()
            pltpu.sync_copy(sh_hist, exch_v)

            def gbody(g, _):
                acc = zeros16
                for t in range(NT):
                    acc = acc + exch_v[t, pl.ds(g * 16, 16)]
                gtot_v[pl.ds(g * 16, 16)] = acc
                return 0
            lax.fori_loop(0, 16, gbody, 0)
            plsc.subcore_barrier()

            def wbody(g, carry):
                cum, bstar, ngt_above = carry
                gbase = 256 - (g + 1) * 16
                row = gtot_v[pl.ds(gbase, 16)]
                rv = lax.rev(row, (0,))
                cs = plsc.cumsum(rv) + cum
                newcum = jnp.max(cs)
                found = jnp.logical_and(newcum >= need, cum < need)
                j = jnp.max(plsc.all_reduce_ffs(cs >= need))
                excl = cs - rv
                prevj = jnp.max(jnp.where(lanes == j, excl, 0))
                bstar = jnp.where(found, gbase + 15 - j, bstar)
                ngt_above = jnp.where(found, prevj, ngt_above)
                return (newcum, bstar, ngt_above)
            _, bstar, ngt_above = lax.fori_loop(
                0, 16, wbody, (jnp.int32(0), jnp.int32(0), jnp.int32(0)))
            need = need - ngt_above
            prefix = prefix | lax.shift_left(bstar, shift)

        tstar = prefix

        def cbody(i, carry):
            ngt_vec, neq_vec = carry
            k = lax.bitcast_convert_type(w_v[pl.ds(i * 16, 16)], jnp.int32)
            ngt_vec = ngt_vec + plsc.all_reduce_population_count(k > tstar)
            neq_vec = neq_vec + plsc.all_reduce_population_count(k == tstar)
            return (ngt_vec, neq_vec)
        ngt_vec, neq_vec = lax.fori_loop(0, NVR, cbody, (zeros16, zeros16))
        ngt = jnp.max(ngt_vec)
        neq = jnp.max(neq_vec)

        crow_v[...] = jnp.where(lanes == 0, ngt, jnp.where(lanes == 1, neq, 0))
        pltpu.sync_copy(crow_v, sh_cnt.at[pl.ds(tid * 16, 16)])
        plsc.subcore_barrier()
        pltpu.sync_copy(sh_cnt, cnt_v)

        gcol = plsc.load_gather(cnt_v, [lanes * 16])
        ecol = plsc.load_gather(cnt_v, [lanes * 16 + 1])
        gt_all = jnp.sum(gcol)
        quota_eq = jnp.int32(K) - gt_all
        ec_excl = plsc.cumsum(ecol) - ecol
        take_vec = jnp.clip(quota_eq - ec_excl, 0, ecol)
        before = lanes < tid
        woff = jnp.sum(jnp.where(before, gcol + take_vec, 0))
        my_take = jnp.max(jnp.where(lanes == tid, take_vec, 0))
        my_nsel = ngt + my_take
        plsc.subcore_barrier()

        def pbody(i, carry):
            off, eqseen = carry
            k = lax.bitcast_convert_type(w_v[pl.ds(i * 16, 16)], jnp.int32)
            gt = k > tstar
            eq = k == tstar
            eqrank = plsc.cumsum(eq.astype(jnp.int32)) + eqseen
            sel = jnp.logical_or(gt, jnp.logical_and(eq, eqrank <= my_take))
            gv = tid * TPT + i * 16 + lanes
            plsc.store_compressed(idxsel_v.at[pl.ds(off, 16)], gv, mask=sel)
            off = off + jnp.max(plsc.all_reduce_population_count(sel))
            eqseen = eqseen + jnp.max(plsc.all_reduce_population_count(eq))
            return (off, eqseen)
        off, _ = lax.fori_loop(0, NVR, pbody, (jnp.int32(0), jnp.int32(0)))
        idxsel_v[pl.ds(off, 16)] = jnp.full((16,), tid * TPT, jnp.int32)

        nchunks = (my_nsel + 15) // 16
        dump_base = B * K + (cid * NT + tid) * 16

        def chunk_body(ch, _):
            vidx = idxsel_v[pl.ds(ch * 16, 16)]
            sidx_v[...] = vidx + b * V
            cp_g = pltpu.async_copy(e_hbm.at[sidx_v], grow_v, sem_g)
            cp_s = pltpu.async_copy(s_hbm.at[sidx_v], sdst_v, sem_s)
            cp_g.wait()
            cp_s.wait()

            svec = sdst_v[...]
            for cc in range(16):
                plsc.store_scatter(sexp_v, [lanes * 16 + cc], svec)

            nrem = my_nsel - ch * 16
            rowbase = b * K + woff + ch * 16
            cps = []
            for j in range(16):
                row = jnp.where(j < nrem, rowbase + j, dump_base + j)
                cps.append(pltpu.async_copy(
                    grow_v.at[j],
                    he_hbm.at[pl.ds(row * F, F)], sem_r))
                cps.append(pltpu.async_copy(
                    sexp_v.at[pl.ds(j * 16, 16)],
                    selk_hbm.at[pl.ds(row * 16, 16)], sem_r))
            for cp in cps:
                cp.wait()
            return 0
        lax.fori_loop(0, nchunks, chunk_body, 0)
        plsc.subcore_barrier()


def _sc2(weights, sel0, emb_rows):
    rows = B * K + PADROWS
    mesh = plsc.VectorSubcoreMesh(core_axis_name="c", subcore_axis_name="s", num_cores=NCORES, num_subcores=NT)
    f = pl.kernel(
        _sc2_body,
        out_type=[
            jax.ShapeDtypeStruct((rows * F,), jnp.float32),
            jax.ShapeDtypeStruct((rows * 16,), jnp.float32),
        ],
        mesh=mesh,
        compiler_params=pltpu.CompilerParams(needs_layout_passes=False),
        scratch_types=[
            pltpu.VMEM((TPT,), jnp.float32),
            pltpu.VMEM((4096,), jnp.int32),
            pltpu.VMEM((256,), jnp.int32),
            pltpu.VMEM((256,), jnp.int32),
            pltpu.VMEM((256,), jnp.int32),
            pltpu.VMEM((NT, 256), jnp.int32),
            pltpu.VMEM((NT * 16,), jnp.int32),
            pltpu.VMEM((K + 32,), jnp.int32),
            pltpu.VMEM((16, F), jnp.float32),
            pltpu.VMEM((16,), jnp.int32),
            pltpu.VMEM((16,), jnp.float32),
            pltpu.VMEM((256,), jnp.float32),
            pltpu.VMEM((16,), jnp.int32),
            pltpu.VMEM_SHARED((NT, 256), jnp.int32),
            pltpu.VMEM_SHARED((NT * 16,), jnp.int32),
            pltpu.SemaphoreType.DMA,
            pltpu.SemaphoreType.DMA,
            pltpu.SemaphoreType.DMA,
        ],
    )
    he_flat, selk_flat = f(weights.reshape(-1), sel0.reshape(-1), emb_rows)
    he = he_flat.reshape(rows, F)[:B * K]
    selk = selk_flat.reshape(rows, 16)[:B * K]
    return he, selk


def _topk_gather_placeholder(weights, sel0, emb):
    _, idx = lax.top_k(weights, K)
    he = jnp.take_along_axis(
        emb.reshape(B, F, V), idx[:, None, :], axis=2)
    he = jnp.transpose(he, (0, 2, 1)).reshape(B * K, F)
    selk = jnp.take_along_axis(sel0, idx, axis=1).reshape(B * K, 1)
    selk = jnp.broadcast_to(selk, (B * K, 16))
    return he, selk


def kernel(proba, y, embeddings, average_representations):
    proba = proba.reshape(B, C, V)
    y = y.reshape(B, C, V)
    embeddings = embeddings.reshape(B, F, V)
    sums, counts, weights, sel0, emb_t = _tc1(proba, y, embeddings)
    he, selk = _sc2(weights, sel0, emb_t.reshape(B * V, F))
    return _tc3(he, selk, sums, counts, average_representations)


# speedup vs baseline: 2.3276x; 1.0006x over previous
import functools

import jax
import jax.numpy as jnp
from jax import lax
from jax.experimental import pallas as pl
from jax.experimental.pallas import tpu as pltpu
from jax.experimental.pallas import tpu_sc as plsc

C = 8
F = 128
B = 4
K = 512
V = 65536
THETA = 0.9
TAU = 0.1

VB = 4096


def _tc1_body(proba_ref, y_ref, emb_ref, sums_ref, counts_ref, w_ref, sel_ref, et_ref):
    b = pl.program_id(0)
    v = pl.program_id(1)

    yf = (y_ref[0] > 0).astype(jnp.float32)
    emb = emb_ref[0]

    @pl.when(jnp.logical_and(b == 0, v == 0))
    def _init():
        sums_ref[...] = jnp.zeros_like(sums_ref)
        counts_ref[...] = jnp.zeros_like(counts_ref)

    sums_ref[...] += lax.dot_general(
        yf, emb, (((1,), (1,)), ((), ())), preferred_element_type=jnp.float32)
    counts_ref[...] += jnp.sum(yf, axis=1, keepdims=True)

    p = proba_ref[0]
    w = p[0]
    for c in range(1, C):
        w = w * p[c]
    w_ref[0] = w.reshape(VB // 128, 128)

    y0 = yf[0]
    ysum = jnp.sum(yf, axis=0)
    sel_ref[0] = jnp.where(jnp.logical_or(y0 > 0.0, ysum == 0.0),
                           1.0, 0.0).reshape(VB // 128, 128)

    et_ref[0] = emb.T


def _tc1(proba, y, emb):
    grid = (B, V // VB)
    return pl.pallas_call(
        _tc1_body,
        grid=grid,
        in_specs=[
            pl.BlockSpec((1, C, VB), lambda b, v: (b, 0, v)),
            pl.BlockSpec((1, C, VB), lambda b, v: (b, 0, v)),
            pl.BlockSpec((1, F, VB), lambda b, v: (b, 0, v)),
        ],
        out_specs=[
            pl.BlockSpec((C, F), lambda b, v: (0, 0)),
            pl.BlockSpec((C, 1), lambda b, v: (0, 0)),
            pl.BlockSpec((1, VB // 128, 128), lambda b, v: (b * (V // VB) + v, 0, 0)),
            pl.BlockSpec((1, VB // 128, 128), lambda b, v: (b * (V // VB) + v, 0, 0)),
            pl.BlockSpec((1, VB, F), lambda b, v: (b, v, 0)),
        ],
        out_shape=[
            jax.ShapeDtypeStruct((C, F), jnp.float32),
            jax.ShapeDtypeStruct((C, 1), jnp.float32),
            jax.ShapeDtypeStruct((B * V // VB, VB // 128, 128), jnp.float32),
            jax.ShapeDtypeStruct((B * V // VB, VB // 128, 128), jnp.float32),
            jax.ShapeDtypeStruct((B, V, F), jnp.float32),
        ],
    )(proba, y, emb)


def _tc3_body(he_ref, selk_ref, sums_ref, counts_ref, avg0_ref, out_ref):
    counts = counts_ref[...]
    sums = sums_ref[...]
    avg0 = avg0_ref[0]
    means = sums / jnp.maximum(counts, 1.0)
    avg = jnp.where(counts > 0.0, avg0 * (1.0 - THETA) + means * THETA, avg0)

    he = he_ref[...]
    acc = jnp.zeros_like(he)
    for c in range(1, C):
        acc += jnp.exp(he * (avg[c][None, :] / TAU))
    rows = jnp.sum(jnp.log(acc), axis=1, keepdims=True)
    rows = rows * selk_ref[...][:, :1]
    seg = (lax.broadcasted_iota(jnp.int32, (B * K, B), 0) // K
           == lax.broadcasted_iota(jnp.int32, (B * K, B), 1)).astype(jnp.float32)
    out_ref[...] = -lax.dot_general(
        rows, seg, (((0,), (0,)), ((), ())),
        preferred_element_type=jnp.float32)


def _tc3(he, selk, sums, counts, avg0):
    return pl.pallas_call(
        _tc3_body,
        out_shape=jax.ShapeDtypeStruct((1, B), jnp.float32),
    )(he, selk, sums, counts, avg0).reshape(B)


NT = 16
NCORES = 2
TPT = V // NT
NVR = TPT // 16
PADROWS = NCORES * NT * 16


def _sc2_body(w_hbm, s_hbm, e_hbm, he_hbm, selk_hbm,
              w_v, hist_v, tot_v, gtot_v, ctmp_v, exch_v, cnt_v, idxsel_v,
              grow_v, sidx_v, sdst_v, sexp_v, crow_v,
              sh_hist, sh_cnt, sem_g, sem_s, sem_r):
    cid = lax.axis_index("c")
    tid = lax.axis_index("s")
    lanes = lax.iota(jnp.int32, 16)
    ones16 = jnp.ones((16,), jnp.int32)
    zeros16 = jnp.zeros((16,), jnp.int32)

    for bi in range(B // NCORES):
        b = cid + NCORES * bi
        base = b * V + tid * TPT
        pltpu.sync_copy(w_hbm.at[pl.ds(base, TPT)], w_v)

        prefix = jnp.int32(0)
        need = jnp.int32(K)
        for level in range(4):
            shift = 24 - 8 * level

            def zbody(i, _):
                hist_v[pl.ds(i * 16, 16)] = zeros16
                return 0
            lax.fori_loop(0, NVR, zbody, 0)

            def hbody(i, _):
                k = lax.bitcast_convert_type(w_v[pl.ds(i * 16, 16)], jnp.int32)
                byte = lax.shift_right_logical(k, shift) & 0xFF
                slot = byte * 16 + lanes
                if level == 0:
                    plsc.addupdate_scatter(hist_v, [slot], ones16)
                else:
                    cand = lax.shift_right_logical(k, shift + 8) == \
                        lax.shift_right_logical(prefix, shift + 8)
                    plsc.addupdate_scatter(hist_v, [slot], ones16, mask=cand)
                return 0
            lax.fori_loop(0, NVR, hbody, 0)

            def tbody(g, _):
                for r in range(16):
                    row = hist_v[pl.ds((g * 16 + r) * 16, 16)]
                    ctmp_v[pl.ds(r * 16, 16)] = plsc.cumsum(row)
                tot_v[pl.ds(g * 16, 16)] = plsc.load_gather(
                    ctmp_v, [lanes * 16 + 15])
                return 0
            lax.fori_loop(0, 16, tbody, 0)

            pltpu.sync_copy(tot_v, sh_hist.at[tid])
            plsc.subcore_barrier()
            pltpu.sync_copy(sh_hist, exch_v)

            def gbody(g, _):
                acc = zeros16
                for t in range(NT):
                    acc = acc + exch_v[t, pl.ds(g * 16, 16)]
                gtot_v[pl.ds(g * 16, 16)] = acc
                return 0
            lax.fori_loop(0, 16, gbody, 0)
            plsc.subcore_barrier()

            def wbody(g, carry):
                cum, bstar, ngt_above = carry
                gbase = 256 - (g + 1) * 16
                row = gtot_v[pl.ds(gbase, 16)]
                rv = lax.rev(row, (0,))
                cs = plsc.cumsum(rv) + cum
                newcum = jnp.max(cs)
                found = jnp.logical_and(newcum >= need, cum < need)
                j = jnp.max(plsc.all_reduce_ffs(cs >= need))
                excl = cs - rv
                prevj = jnp.max(jnp.where(lanes == j, excl, 0))
                bstar = jnp.where(found, gbase + 15 - j, bstar)
                ngt_above = jnp.where(found, prevj, ngt_above)
                return (newcum, bstar, ngt_above)
            _, bstar, ngt_above = lax.fori_loop(
                0, 16, wbody, (jnp.int32(0), jnp.int32(0), jnp.int32(0)))
            need = need - ngt_above
            prefix = prefix | lax.shift_left(bstar, shift)

        tstar = prefix

        def cbody(i, carry):
            ngt_vec, neq_vec = carry
            k = lax.bitcast_convert_type(w_v[pl.ds(i * 16, 16)], jnp.int32)
            ngt_vec = ngt_vec + plsc.all_reduce_population_count(k > tstar)
            neq_vec = neq_vec + plsc.all_reduce_population_count(k == tstar)
            return (ngt_vec, neq_vec)
        ngt_vec, neq_vec = lax.fori_loop(0, NVR, cbody, (zeros16, zeros16))
        ngt = jnp.max(ngt_vec)
        neq = jnp.max(neq_vec)

        crow_v[...] = jnp.where(lanes == 0, ngt, jnp.where(lanes == 1, neq, 0))
        pltpu.sync_copy(crow_v, sh_cnt.at[pl.ds(tid * 16, 16)])
        plsc.subcore_barrier()
        pltpu.sync_copy(sh_cnt, cnt_v)

        gcol = plsc.load_gather(cnt_v, [lanes * 16])
        ecol = plsc.load_gather(cnt_v, [lanes * 16 + 1])
        gt_all = jnp.sum(gcol)
        quota_eq = jnp.int32(K) - gt_all
        ec_excl = plsc.cumsum(ecol) - ecol
        take_vec = jnp.clip(quota_eq - ec_excl, 0, ecol)
        before = lanes < tid
        woff = jnp.sum(jnp.where(before, gcol + take_vec, 0))
        my_take = jnp.max(jnp.where(lanes == tid, take_vec, 0))
        my_nsel = ngt + my_take
        plsc.subcore_barrier()

        def pbody(i, carry):
            off, eqseen = carry
            k = lax.bitcast_convert_type(w_v[pl.ds(i * 16, 16)], jnp.int32)
            gt = k > tstar
            eq = k == tstar
            eqrank = plsc.cumsum(eq.astype(jnp.int32)) + eqseen
            sel = jnp.logical_or(gt, jnp.logical_and(eq, eqrank <= my_take))
            gv = tid * TPT + i * 16 + lanes
            plsc.store_compressed(idxsel_v.at[pl.ds(off, 16)], gv, mask=sel)
            off = off + jnp.max(plsc.all_reduce_population_count(sel))
            eqseen = eqseen + jnp.max(plsc.all_reduce_population_count(eq))
            return (off, eqseen)
        off, _ = lax.fori_loop(0, NVR, pbody, (jnp.int32(0), jnp.int32(0)))
        idxsel_v[pl.ds(off, 16)] = jnp.full((16,), tid * TPT, jnp.int32)

        nchunks = (my_nsel + 15) // 16
        dump_base = B * K + (cid * NT + tid) * 16

        def chunk_body(ch, _):
            vidx = idxsel_v[pl.ds(ch * 16, 16)]
            sidx_v[...] = vidx + b * V
            cp_g = pltpu.async_copy(e_hbm.at[sidx_v], grow_v, sem_g)
            cp_s = pltpu.async_copy(s_hbm.at[sidx_v], sdst_v, sem_s)
            cp_g.wait()
            cp_s.wait()

            svec = sdst_v[...]
            for cc in range(16):
                plsc.store_scatter(sexp_v, [lanes * 16 + cc], svec)

            nrem = my_nsel - ch * 16
            rowbase = b * K + woff + ch * 16
            cps = []
            for j in range(16):
                row = jnp.where(j < nrem, rowbase + j, dump_base + j)
                cps.append(pltpu.async_copy(
                    grow_v.at[j],
                    he_hbm.at[pl.ds(row * F, F)], sem_r))
                cps.append(pltpu.async_copy(
                    sexp_v.at[pl.ds(j * 16, 16)],
                    selk_hbm.at[pl.ds(row * 16, 16)], sem_r))
            for cp in cps:
                cp.wait()
            return 0
        lax.fori_loop(0, nchunks, chunk_body, 0)
        plsc.subcore_barrier()


def _sc2(weights, sel0, emb_rows):
    rows = B * K + PADROWS
    mesh = plsc.VectorSubcoreMesh(core_axis_name="c", subcore_axis_name="s", num_cores=NCORES, num_subcores=NT)
    f = pl.kernel(
        _sc2_body,
        out_type=[
            jax.ShapeDtypeStruct((rows * F,), jnp.float32),
            jax.ShapeDtypeStruct((rows * 16,), jnp.float32),
        ],
        mesh=mesh,
        compiler_params=pltpu.CompilerParams(needs_layout_passes=False),
        scratch_types=[
            pltpu.VMEM((TPT,), jnp.float32),
            pltpu.VMEM((4096,), jnp.int32),
            pltpu.VMEM((256,), jnp.int32),
            pltpu.VMEM((256,), jnp.int32),
            pltpu.VMEM((256,), jnp.int32),
            pltpu.VMEM((NT, 256), jnp.int32),
            pltpu.VMEM((NT * 16,), jnp.int32),
            pltpu.VMEM((K + 32,), jnp.int32),
            pltpu.VMEM((16, F), jnp.float32),
            pltpu.VMEM((16,), jnp.int32),
            pltpu.VMEM((16,), jnp.float32),
            pltpu.VMEM((256,), jnp.float32),
            pltpu.VMEM((16,), jnp.int32),
            pltpu.VMEM_SHARED((NT, 256), jnp.int32),
            pltpu.VMEM_SHARED((NT * 16,), jnp.int32),
            pltpu.SemaphoreType.DMA,
            pltpu.SemaphoreType.DMA,
            pltpu.SemaphoreType.DMA,
        ],
    )
    he_flat, selk_flat = f(weights.reshape(-1), sel0.reshape(-1), emb_rows)
    he = he_flat.reshape(rows, F)[:B * K]
    selk = selk_flat.reshape(rows, 16)[:B * K]
    return he, selk


def _topk_gather_placeholder(weights, sel0, emb):
    _, idx = lax.top_k(weights, K)
    he = jnp.take_along_axis(
        emb.reshape(B, F, V), idx[:, None, :], axis=2)
    he = jnp.transpose(he, (0, 2, 1)).reshape(B * K, F)
    selk = jnp.take_along_axis(sel0, idx, axis=1).reshape(B * K, 1)
    selk = jnp.broadcast_to(selk, (B * K, 16))
    return he, selk


def kernel(proba, y, embeddings, average_representations):
    proba = proba.reshape(B, C, V)
    y = y.reshape(B, C, V)
    embeddings = embeddings.reshape(B, F, V)
    sums, counts, weights, sel0, emb_t = _tc1(proba, y, embeddings)
    he, selk = _sc2(weights, sel0, emb_t.reshape(B * V, F))
    return _tc3(he, selk, sums, counts, average_representations)


# speedup vs baseline: 3.2441x; 1.3938x over previous
import functools

import jax
import jax.numpy as jnp
from jax import lax
from jax.experimental import pallas as pl
from jax.experimental.pallas import tpu as pltpu
from jax.experimental.pallas import tpu_sc as plsc

C = 8
F = 128
B = 4
K = 512
V = 65536
THETA = 0.9
TAU = 0.1

VB = 4096


def _tc1_body(proba_ref, y_ref, emb_ref, sums_ref, counts_ref, w_ref, sel_ref, et_ref):
    b = pl.program_id(0)
    v = pl.program_id(1)
    HB = VB // 256

    yf = (y_ref[0] > 0).astype(jnp.float32)
    emb = emb_ref[0]
    p = proba_ref[0]

    @pl.when(jnp.logical_and(b == 0, v == 0))
    def _init():
        sums_ref[...] = jnp.zeros_like(sums_ref)
        counts_ref[...] = jnp.zeros_like(counts_ref)

    acc = sums_ref[...]
    for h in range(HB):
        acc = acc + lax.dot_general(
            yf[:, h, :], emb[:, h, :], (((1,), (1,)), ((), ())),
            preferred_element_type=jnp.float32)
    sums_ref[...] = acc
    counts_ref[...] += jnp.sum(jnp.sum(yf, axis=2), axis=1, keepdims=True)

    w = p[0]
    for c in range(1, C):
        w = w * p[c]
    y0 = yf[0]
    ysum = jnp.sum(yf, axis=0)
    sel = jnp.where(jnp.logical_or(y0 > 0.0, ysum == 0.0), 1.0, 0.0)
    for h in range(HB):
        w_ref[0, pl.ds(h * 2, 2), :] = w[h].reshape(2, 128)
        sel_ref[0, pl.ds(h * 2, 2), :] = sel[h].reshape(2, 128)
        et_ref[0, pl.ds(h * 256, 256), :] = emb[:, h, :].T


def _tc1(proba, y, emb):
    grid = (B, V // VB)
    return pl.pallas_call(
        _tc1_body,
        grid=grid,
        in_specs=[
            pl.BlockSpec((1, C, VB // 256, 256), lambda b, v: (b, 0, v, 0)),
            pl.BlockSpec((1, C, VB // 256, 256), lambda b, v: (b, 0, v, 0)),
            pl.BlockSpec((1, F, VB // 256, 256), lambda b, v: (b, 0, v, 0)),
        ],
        out_specs=[
            pl.BlockSpec((C, F), lambda b, v: (0, 0)),
            pl.BlockSpec((C, 1), lambda b, v: (0, 0)),
            pl.BlockSpec((1, VB // 128, 128), lambda b, v: (b * (V // VB) + v, 0, 0)),
            pl.BlockSpec((1, VB // 128, 128), lambda b, v: (b * (V // VB) + v, 0, 0)),
            pl.BlockSpec((1, VB, F), lambda b, v: (b, v, 0)),
        ],
        out_shape=[
            jax.ShapeDtypeStruct((C, F), jnp.float32),
            jax.ShapeDtypeStruct((C, 1), jnp.float32),
            jax.ShapeDtypeStruct((B * V // VB, VB // 128, 128), jnp.float32),
            jax.ShapeDtypeStruct((B * V // VB, VB // 128, 128), jnp.float32),
            jax.ShapeDtypeStruct((B, V, F), jnp.float32),
        ],
    )(proba, y, emb)


def _tc3_body(he_ref, selk_ref, sums_ref, counts_ref, avg0_ref, out_ref):
    counts = counts_ref[...]
    sums = sums_ref[...]
    avg0 = avg0_ref[0]
    means = sums / jnp.maximum(counts, 1.0)
    avg = jnp.where(counts > 0.0, avg0 * (1.0 - THETA) + means * THETA, avg0)

    he = he_ref[...]
    acc = jnp.zeros_like(he)
    for c in range(1, C):
        acc += jnp.exp(he * (avg[c][None, :] / TAU))
    rows = jnp.sum(jnp.log(acc), axis=1, keepdims=True)
    rows = rows * selk_ref[...][:, :1]
    seg = (lax.broadcasted_iota(jnp.int32, (B * K, B), 0) // K
           == lax.broadcasted_iota(jnp.int32, (B * K, B), 1)).astype(jnp.float32)
    out_ref[...] = -lax.dot_general(
        rows, seg, (((0,), (0,)), ((), ())),
        preferred_element_type=jnp.float32)


def _tc3(he, selk, sums, counts, avg0):
    return pl.pallas_call(
        _tc3_body,
        out_shape=jax.ShapeDtypeStruct((1, B), jnp.float32),
    )(he, selk, sums, counts, avg0).reshape(B)


NT = 16
NCORES = 2
TPT = V // NT
NVR = TPT // 16
PADROWS = NCORES * NT * 16


def _sc2_body(w_hbm, s_hbm, e_hbm, he_hbm, selk_hbm,
              w_v, hist_v, tot_v, gtot_v, ctmp_v, exch_v, cnt_v, idxsel_v,
              grow_v, sidx_v, sdst_v, sexp_v, crow_v,
              sh_hist, sh_cnt, sem_g, sem_s, sem_r):
    cid = lax.axis_index("c")
    tid = lax.axis_index("s")
    lanes = lax.iota(jnp.int32, 16)
    ones16 = jnp.ones((16,), jnp.int32)
    zeros16 = jnp.zeros((16,), jnp.int32)

    for bi in range(B // NCORES):
        b = cid + NCORES * bi
        base = b * V + tid * TPT
        pltpu.sync_copy(w_hbm.at[pl.ds(base, TPT)], w_v)

        prefix = jnp.int32(0)
        need = jnp.int32(K)
        for level in range(4):
            shift = 24 - 8 * level

            def zbody(i, _):
                hist_v[pl.ds(i * 16, 16)] = zeros16
                return 0
            lax.fori_loop(0, NVR, zbody, 0)

            def hbody(i, _):
                k = lax.bitcast_convert_type(w_v[pl.ds(i * 16, 16)], jnp.int32)
                byte = lax.shift_right_logical(k, shift) & 0xFF
                slot = byte * 16 + lanes
                if level == 0:
                    plsc.addupdate_scatter(hist_v, [slot], ones16)
                else:
                    cand = lax.shift_right_logical(k, shift + 8) == \
                        lax.shift_right_logical(prefix, shift + 8)
                    plsc.addupdate_scatter(hist_v, [slot], ones16, mask=cand)
                return 0
            lax.fori_loop(0, NVR, hbody, 0)

            def tbody(g, _):
                for r in range(16):
                    row = hist_v[pl.ds((g * 16 + r) * 16, 16)]
                    ctmp_v[pl.ds(r * 16, 16)] = plsc.cumsum(row)
                tot_v[pl.ds(g * 16, 16)] = plsc.load_gather(
                    ctmp_v, [lanes * 16 + 15])
                return 0
            lax.fori_loop(0, 16, tbody, 0)

            pltpu.sync_copy(tot_v, sh_hist.at[tid])
            plsc.subcore_barrier()
            pltpu.sync_copy(sh_hist, exch_v)

            def gbody(g, _):
                acc = zeros16
                for t in range(NT):
                    acc = acc + exch_v[t, pl.ds(g * 16, 16)]
                gtot_v[pl.ds(g * 16, 16)] = acc
                return 0
            lax.fori_loop(0, 16, gbody, 0)
            plsc.subcore_barrier()

            def wbody(g, carry):
                cum, bstar, ngt_above = carry
                gbase = 256 - (g + 1) * 16
                row = gtot_v[pl.ds(gbase, 16)]
                rv = lax.rev(row, (0,))
                cs = plsc.cumsum(rv) + cum
                newcum = jnp.max(cs)
                found = jnp.logical_and(newcum >= need, cum < need)
                j = jnp.max(plsc.all_reduce_ffs(cs >= need))
                excl = cs - rv
                prevj = jnp.max(jnp.where(lanes == j, excl, 0))
                bstar = jnp.where(found, gbase + 15 - j, bstar)
                ngt_above = jnp.where(found, prevj, ngt_above)
                return (newcum, bstar, ngt_above)
            _, bstar, ngt_above = lax.fori_loop(
                0, 16, wbody, (jnp.int32(0), jnp.int32(0), jnp.int32(0)))
            need = need - ngt_above
            prefix = prefix | lax.shift_left(bstar, shift)

        tstar = prefix

        def cbody(i, carry):
            ngt_vec, neq_vec = carry
            k = lax.bitcast_convert_type(w_v[pl.ds(i * 16, 16)], jnp.int32)
            ngt_vec = ngt_vec + plsc.all_reduce_population_count(k > tstar)
            neq_vec = neq_vec + plsc.all_reduce_population_count(k == tstar)
            return (ngt_vec, neq_vec)
        ngt_vec, neq_vec = lax.fori_loop(0, NVR, cbody, (zeros16, zeros16))
        ngt = jnp.max(ngt_vec)
        neq = jnp.max(neq_vec)

        crow_v[...] = jnp.where(lanes == 0, ngt, jnp.where(lanes == 1, neq, 0))
        pltpu.sync_copy(crow_v, sh_cnt.at[pl.ds(tid * 16, 16)])
        plsc.subcore_barrier()
        pltpu.sync_copy(sh_cnt, cnt_v)

        gcol = plsc.load_gather(cnt_v, [lanes * 16])
        ecol = plsc.load_gather(cnt_v, [lanes * 16 + 1])
        gt_all = jnp.sum(gcol)
        quota_eq = jnp.int32(K) - gt_all
        ec_excl = plsc.cumsum(ecol) - ecol
        take_vec = jnp.clip(quota_eq - ec_excl, 0, ecol)
        before = lanes < tid
        woff = jnp.sum(jnp.where(before, gcol + take_vec, 0))
        my_take = jnp.max(jnp.where(lanes == tid, take_vec, 0))
        my_nsel = ngt + my_take
        plsc.subcore_barrier()

        def pbody(i, carry):
            off, eqseen = carry
            k = lax.bitcast_convert_type(w_v[pl.ds(i * 16, 16)], jnp.int32)
            gt = k > tstar
            eq = k == tstar
            eqrank = plsc.cumsum(eq.astype(jnp.int32)) + eqseen
            sel = jnp.logical_or(gt, jnp.logical_and(eq, eqrank <= my_take))
            gv = tid * TPT + i * 16 + lanes
            plsc.store_compressed(idxsel_v.at[pl.ds(off, 16)], gv, mask=sel)
            off = off + jnp.max(plsc.all_reduce_population_count(sel))
            eqseen = eqseen + jnp.max(plsc.all_reduce_population_count(eq))
            return (off, eqseen)
        off, _ = lax.fori_loop(0, NVR, pbody, (jnp.int32(0), jnp.int32(0)))
        idxsel_v[pl.ds(off, 16)] = jnp.full((16,), tid * TPT, jnp.int32)

        nchunks = (my_nsel + 15) // 16
        dump_base = B * K + (cid * NT + tid) * 16

        def chunk_body(ch, _):
            vidx = idxsel_v[pl.ds(ch * 16, 16)]
            sidx_v[...] = vidx + b * V
            cp_g = pltpu.async_copy(e_hbm.at[sidx_v], grow_v, sem_g)
            cp_s = pltpu.async_copy(s_hbm.at[sidx_v], sdst_v, sem_s)
            cp_g.wait()
            cp_s.wait()

            svec = sdst_v[...]
            for cc in range(16):
                plsc.store_scatter(sexp_v, [lanes * 16 + cc], svec)

            nrem = my_nsel - ch * 16
            rowbase = b * K + woff + ch * 16
            cps = []
            for j in range(16):
                row = jnp.where(j < nrem, rowbase + j, dump_base + j)
                cps.append(pltpu.async_copy(
                    grow_v.at[j],
                    he_hbm.at[pl.ds(row * F, F)], sem_r))
                cps.append(pltpu.async_copy(
                    sexp_v.at[pl.ds(j * 16, 16)],
                    selk_hbm.at[pl.ds(row * 16, 16)], sem_r))
            for cp in cps:
                cp.wait()
            return 0
        lax.fori_loop(0, nchunks, chunk_body, 0)
        plsc.subcore_barrier()


def _sc2(weights, sel0, emb_rows):
    rows = B * K + PADROWS
    mesh = plsc.VectorSubcoreMesh(core_axis_name="c", subcore_axis_name="s", num_cores=NCORES, num_subcores=NT)
    f = pl.kernel(
        _sc2_body,
        out_type=[
            jax.ShapeDtypeStruct((rows * F,), jnp.float32),
            jax.ShapeDtypeStruct((rows * 16,), jnp.float32),
        ],
        mesh=mesh,
        compiler_params=pltpu.CompilerParams(needs_layout_passes=False),
        scratch_types=[
            pltpu.VMEM((TPT,), jnp.float32),
            pltpu.VMEM((4096,), jnp.int32),
            pltpu.VMEM((256,), jnp.int32),
            pltpu.VMEM((256,), jnp.int32),
            pltpu.VMEM((256,), jnp.int32),
            pltpu.VMEM((NT, 256), jnp.int32),
            pltpu.VMEM((NT * 16,), jnp.int32),
            pltpu.VMEM((K + 32,), jnp.int32),
            pltpu.VMEM((16, F), jnp.float32),
            pltpu.VMEM((16,), jnp.int32),
            pltpu.VMEM((16,), jnp.float32),
            pltpu.VMEM((256,), jnp.float32),
            pltpu.VMEM((16,), jnp.int32),
            pltpu.VMEM_SHARED((NT, 256), jnp.int32),
            pltpu.VMEM_SHARED((NT * 16,), jnp.int32),
            pltpu.SemaphoreType.DMA,
            pltpu.SemaphoreType.DMA,
            pltpu.SemaphoreType.DMA,
        ],
    )
    he_flat, selk_flat = f(weights.reshape(-1), sel0.reshape(-1), emb_rows)
    he = he_flat.reshape(rows, F)[:B * K]
    selk = selk_flat.reshape(rows, 16)[:B * K]
    return he, selk


def _topk_gather_placeholder(weights, sel0, emb):
    _, idx = lax.top_k(weights, K)
    he = jnp.take_along_axis(
        emb.reshape(B, F, V), idx[:, None, :], axis=2)
    he = jnp.transpose(he, (0, 2, 1)).reshape(B * K, F)
    selk = jnp.take_along_axis(sel0, idx, axis=1).reshape(B * K, 1)
    selk = jnp.broadcast_to(selk, (B * K, 16))
    return he, selk


def kernel(proba, y, embeddings, average_representations):
    sums, counts, weights, sel0, emb_t = _tc1(proba, y, embeddings)
    he, selk = _sc2(weights, sel0, emb_t.reshape(B * V, F))
    return _tc3(he, selk, sums, counts, average_representations)


# speedup vs baseline: 3.7977x; 1.1706x over previous
import functools

import jax
import jax.numpy as jnp
from jax import lax
from jax.experimental import pallas as pl
from jax.experimental.pallas import tpu as pltpu
from jax.experimental.pallas import tpu_sc as plsc

C = 8
F = 128
B = 4
K = 512
V = 65536
THETA = 0.9
TAU = 0.1

VB = 4096


def _tc1a_body(proba_ref, y_ref, w_ref, sel_ref):
    HB = VB // 256
    yf = (y_ref[0] > 0).astype(jnp.float32)
    p = proba_ref[0]
    w = p[0]
    for c in range(1, C):
        w = w * p[c]
    y0 = yf[0]
    ysum = jnp.sum(yf, axis=0)
    sel = jnp.where(jnp.logical_or(y0 > 0.0, ysum == 0.0), 1.0, 0.0)
    for h in range(HB):
        w_ref[0, pl.ds(h * 2, 2), :] = w[h].reshape(2, 128)
        sel_ref[0, pl.ds(h * 2, 2), :] = sel[h].reshape(2, 128)


def _tc1a(proba, y):
    grid = (B, V // VB)
    return pl.pallas_call(
        _tc1a_body,
        grid=grid,
        in_specs=[
            pl.BlockSpec((1, C, VB // 256, 256), lambda b, v: (b, 0, v, 0)),
            pl.BlockSpec((1, C, VB // 256, 256), lambda b, v: (b, 0, v, 0)),
        ],
        out_specs=[
            pl.BlockSpec((1, VB // 128, 128), lambda b, v: (b * (V // VB) + v, 0, 0)),
            pl.BlockSpec((1, VB // 128, 128), lambda b, v: (b * (V // VB) + v, 0, 0)),
        ],
        out_shape=[
            jax.ShapeDtypeStruct((B * V // VB, VB // 128, 128), jnp.float32),
            jax.ShapeDtypeStruct((B * V // VB, VB // 128, 128), jnp.float32),
        ],
    )(proba, y)


def _tc1b_body(y_ref, emb_ref, sums_ref, counts_ref, et_ref):
    b = pl.program_id(0)
    v = pl.program_id(1)
    HB = VB // 256
    yf = (y_ref[0] > 0).astype(jnp.float32)
    emb = emb_ref[0]

    @pl.when(jnp.logical_and(b == 0, v == 0))
    def _init():
        sums_ref[...] = jnp.zeros_like(sums_ref)
        counts_ref[...] = jnp.zeros_like(counts_ref)

    acc = sums_ref[...]
    for h in range(HB):
        acc = acc + lax.dot_general(
            yf[:, h, :], emb[:, h, :], (((1,), (1,)), ((), ())),
            preferred_element_type=jnp.float32)
    sums_ref[...] = acc
    counts_ref[...] += jnp.sum(jnp.sum(yf, axis=2), axis=1, keepdims=True)
    for h in range(HB):
        et_ref[0, pl.ds(h * 256, 256), :] = emb[:, h, :].T


def _tc1b(y, emb):
    grid = (B, V // VB)
    return pl.pallas_call(
        _tc1b_body,
        grid=grid,
        in_specs=[
            pl.BlockSpec((1, C, VB // 256, 256), lambda b, v: (b, 0, v, 0)),
            pl.BlockSpec((1, F, VB // 256, 256), lambda b, v: (b, 0, v, 0)),
        ],
        out_specs=[
            pl.BlockSpec((C, F), lambda b, v: (0, 0)),
            pl.BlockSpec((C, 1), lambda b, v: (0, 0)),
            pl.BlockSpec((1, VB, F), lambda b, v: (b, v, 0)),
        ],
        out_shape=[
            jax.ShapeDtypeStruct((C, F), jnp.float32),
            jax.ShapeDtypeStruct((C, 1), jnp.float32),
            jax.ShapeDtypeStruct((B, V, F), jnp.float32),
        ],
    )(y, emb)


def _tc3_body(he_ref, selk_ref, sums_ref, counts_ref, avg0_ref, out_ref):
    counts = counts_ref[...]
    sums = sums_ref[...]
    avg0 = avg0_ref[0]
    means = sums / jnp.maximum(counts, 1.0)
    avg = jnp.where(counts > 0.0, avg0 * (1.0 - THETA) + means * THETA, avg0)

    he = he_ref[...]
    acc = jnp.zeros_like(he)
    for c in range(1, C):
        acc += jnp.exp(he * (avg[c][None, :] / TAU))
    rows = jnp.sum(jnp.log(acc), axis=1, keepdims=True)
    rows = rows * selk_ref[...][:, :1]
    seg = (lax.broadcasted_iota(jnp.int32, (B * K, B), 0) // K
           == lax.broadcasted_iota(jnp.int32, (B * K, B), 1)).astype(jnp.float32)
    out_ref[...] = -lax.dot_general(
        rows, seg, (((0,), (0,)), ((), ())),
        preferred_element_type=jnp.float32)


def _tc3(he, selk, sums, counts, avg0):
    return pl.pallas_call(
        _tc3_body,
        out_shape=jax.ShapeDtypeStruct((1, B), jnp.float32),
    )(he, selk, sums, counts, avg0).reshape(B)


NT = 16
NCORES = 2
TPT = V // NT
NVR = TPT // 16
PADROWS = NCORES * NT * 16


def _sc2_body(w_hbm, s_hbm, idx_hbm, selk_hbm,
              w_v, hist_v, tot_v, gtot_v, ctmp_v, exch_v, cnt_v, idxsel_v,
              sidx_v, sdst_v, sexp_v, iexp_v, crow_v,
              sh_hist, sh_cnt, sem_s, sem_r):
    cid = lax.axis_index("c")
    tid = lax.axis_index("s")
    lanes = lax.iota(jnp.int32, 16)
    ones16 = jnp.ones((16,), jnp.int32)
    zeros16 = jnp.zeros((16,), jnp.int32)

    for bi in range(B // NCORES):
        b = cid + NCORES * bi
        base = b * V + tid * TPT
        pltpu.sync_copy(w_hbm.at[pl.ds(base, TPT)], w_v)

        prefix = jnp.int32(0)
        need = jnp.int32(K)
        for level in range(4):
            shift = 24 - 8 * level

            def zbody(i, _):
                hist_v[pl.ds(i * 16, 16)] = zeros16
                return 0
            lax.fori_loop(0, NVR, zbody, 0)

            def hbody(i, _):
                k = lax.bitcast_convert_type(w_v[pl.ds(i * 16, 16)], jnp.int32)
                byte = lax.shift_right_logical(k, shift) & 0xFF
                slot = byte * 16 + lanes
                if level == 0:
                    plsc.addupdate_scatter(hist_v, [slot], ones16)
                else:
                    cand = lax.shift_right_logical(k, shift + 8) == \
                        lax.shift_right_logical(prefix, shift + 8)
                    plsc.addupdate_scatter(hist_v, [slot], ones16, mask=cand)
                return 0
            lax.fori_loop(0, NVR, hbody, 0)

            def tbody(g, _):
                for r in range(16):
                    row = hist_v[pl.ds((g * 16 + r) * 16, 16)]
                    ctmp_v[pl.ds(r * 16, 16)] = plsc.cumsum(row)
                tot_v[pl.ds(g * 16, 16)] = plsc.load_gather(
                    ctmp_v, [lanes * 16 + 15])
                return 0
            lax.fori_loop(0, 16, tbody, 0)

            pltpu.sync_copy(tot_v, sh_hist.at[tid])
            plsc.subcore_barrier()
            pltpu.sync_copy(sh_hist, exch_v)

            def gbody(g, _):
                acc = zeros16
                for t in range(NT):
                    acc = acc + exch_v[t, pl.ds(g * 16, 16)]
                gtot_v[pl.ds(g * 16, 16)] = acc
                return 0
            lax.fori_loop(0, 16, gbody, 0)
            plsc.subcore_barrier()

            def wbody(g, carry):
                cum, bstar, ngt_above = carry
                gbase = 256 - (g + 1) * 16
                row = gtot_v[pl.ds(gbase, 16)]
                rv = lax.rev(row, (0,))
                cs = plsc.cumsum(rv) + cum
                newcum = jnp.max(cs)
                found = jnp.logical_and(newcum >= need, cum < need)
                j = jnp.max(plsc.all_reduce_ffs(cs >= need))
                excl = cs - rv
                prevj = jnp.max(jnp.where(lanes == j, excl, 0))
                bstar = jnp.where(found, gbase + 15 - j, bstar)
                ngt_above = jnp.where(found, prevj, ngt_above)
                return (newcum, bstar, ngt_above)
            _, bstar, ngt_above = lax.fori_loop(
                0, 16, wbody, (jnp.int32(0), jnp.int32(0), jnp.int32(0)))
            need = need - ngt_above
            prefix = prefix | lax.shift_left(bstar, shift)

        tstar = prefix

        def cbody(i, carry):
            ngt_vec, neq_vec = carry
            k = lax.bitcast_convert_type(w_v[pl.ds(i * 16, 16)], jnp.int32)
            ngt_vec = ngt_vec + plsc.all_reduce_population_count(k > tstar)
            neq_vec = neq_vec + plsc.all_reduce_population_count(k == tstar)
            return (ngt_vec, neq_vec)
        ngt_vec, neq_vec = lax.fori_loop(0, NVR, cbody, (zeros16, zeros16))
        ngt = jnp.max(ngt_vec)
        neq = jnp.max(neq_vec)

        crow_v[...] = jnp.where(lanes == 0, ngt, jnp.where(lanes == 1, neq, 0))
        pltpu.sync_copy(crow_v, sh_cnt.at[pl.ds(tid * 16, 16)])
        plsc.subcore_barrier()
        pltpu.sync_copy(sh_cnt, cnt_v)

        gcol = plsc.load_gather(cnt_v, [lanes * 16])
        ecol = plsc.load_gather(cnt_v, [lanes * 16 + 1])
        gt_all = jnp.sum(gcol)
        quota_eq = jnp.int32(K) - gt_all
        ec_excl = plsc.cumsum(ecol) - ecol
        take_vec = jnp.clip(quota_eq - ec_excl, 0, ecol)
        before = lanes < tid
        woff = jnp.sum(jnp.where(before, gcol + take_vec, 0))
        my_take = jnp.max(jnp.where(lanes == tid, take_vec, 0))
        my_nsel = ngt + my_take
        plsc.subcore_barrier()

        def pbody(i, carry):
            off, eqseen = carry
            k = lax.bitcast_convert_type(w_v[pl.ds(i * 16, 16)], jnp.int32)
            gt = k > tstar
            eq = k == tstar
            eqrank = plsc.cumsum(eq.astype(jnp.int32)) + eqseen
            sel = jnp.logical_or(gt, jnp.logical_and(eq, eqrank <= my_take))
            gv = tid * TPT + i * 16 + lanes
            plsc.store_compressed(idxsel_v.at[pl.ds(off, 16)], gv, mask=sel)
            off = off + jnp.max(plsc.all_reduce_population_count(sel))
            eqseen = eqseen + jnp.max(plsc.all_reduce_population_count(eq))
            return (off, eqseen)
        off, _ = lax.fori_loop(0, NVR, pbody, (jnp.int32(0), jnp.int32(0)))
        idxsel_v[pl.ds(off, 16)] = jnp.full((16,), tid * TPT, jnp.int32)

        nchunks = (my_nsel + 15) // 16
        dump_base = B * K + (cid * NT + tid) * 16

        def chunk_body(ch, _):
            vidx = idxsel_v[pl.ds(ch * 16, 16)]
            sidx_v[...] = vidx + b * V
            cp_s = pltpu.async_copy(s_hbm.at[sidx_v], sdst_v, sem_s)
            cp_s.wait()

            svec = sdst_v[...]
            rvec = sidx_v[...]
            for cc in range(16):
                plsc.store_scatter(sexp_v, [lanes * 16 + cc], svec)
                plsc.store_scatter(iexp_v, [lanes * 16 + cc], rvec)

            nrem = my_nsel - ch * 16
            rowbase = b * K + woff + ch * 16
            cps = []
            for j in range(16):
                row = jnp.where(j < nrem, rowbase + j, dump_base + j)
                cps.append(pltpu.async_copy(
                    iexp_v.at[pl.ds(j * 16, 16)],
                    idx_hbm.at[pl.ds(row * 16, 16)], sem_r))
                cps.append(pltpu.async_copy(
                    sexp_v.at[pl.ds(j * 16, 16)],
                    selk_hbm.at[pl.ds(row * 16, 16)], sem_r))
            for cp in cps:
                cp.wait()
            return 0
        lax.fori_loop(0, nchunks, chunk_body, 0)
        plsc.subcore_barrier()


def _sc2(weights, sel0):
    rows = B * K + PADROWS
    mesh = plsc.VectorSubcoreMesh(core_axis_name="c", subcore_axis_name="s", num_cores=NCORES, num_subcores=NT)
    f = pl.kernel(
        _sc2_body,
        out_type=[
            jax.ShapeDtypeStruct((rows * 16,), jnp.int32),
            jax.ShapeDtypeStruct((rows * 16,), jnp.float32),
        ],
        mesh=mesh,
        compiler_params=pltpu.CompilerParams(needs_layout_passes=False),
        scratch_types=[
            pltpu.VMEM((TPT,), jnp.float32),
            pltpu.VMEM((4096,), jnp.int32),
            pltpu.VMEM((256,), jnp.int32),
            pltpu.VMEM((256,), jnp.int32),
            pltpu.VMEM((256,), jnp.int32),
            pltpu.VMEM((NT, 256), jnp.int32),
            pltpu.VMEM((NT * 16,), jnp.int32),
            pltpu.VMEM((K + 32,), jnp.int32),
            pltpu.VMEM((16,), jnp.int32),
            pltpu.VMEM((16,), jnp.float32),
            pltpu.VMEM((256,), jnp.float32),
            pltpu.VMEM((256,), jnp.int32),
            pltpu.VMEM((16,), jnp.int32),
            pltpu.VMEM_SHARED((NT, 256), jnp.int32),
            pltpu.VMEM_SHARED((NT * 16,), jnp.int32),
            pltpu.SemaphoreType.DMA,
            pltpu.SemaphoreType.DMA,
        ],
    )
    idx_flat, selk_flat = f(weights.reshape(-1), sel0.reshape(-1))
    idx = idx_flat.reshape(rows, 16)[:B * K]
    selk = selk_flat.reshape(rows, 16)[:B * K]
    return idx, selk


ROWS_PER_W = B * K // (NCORES * NT)


def _scg_body(idx_hbm, e_hbm, he_hbm, ibuf_v, rid_v, grow_v, sem_g):
    cid = lax.axis_index("c")
    tid = lax.axis_index("s")
    lanes = lax.iota(jnp.int32, 16)
    wid = tid * NCORES + cid
    rb = wid * ROWS_PER_W
    for g in range(ROWS_PER_W // 16):
        r0 = rb + g * 16
        pltpu.sync_copy(idx_hbm.at[pl.ds(r0 * 16, 256)], ibuf_v)
        rid_v[...] = plsc.load_gather(ibuf_v, [lanes * 16])
        pltpu.async_copy(e_hbm.at[rid_v], grow_v, sem_g).wait()
        pltpu.sync_copy(grow_v, he_hbm.at[pl.ds(r0, 16)])


def _scg(idx, emb_rows):
    mesh = plsc.VectorSubcoreMesh(core_axis_name="c", subcore_axis_name="s", num_cores=NCORES, num_subcores=NT)
    f = pl.kernel(
        _scg_body,
        out_type=[jax.ShapeDtypeStruct((B * K, F), jnp.float32)],
        mesh=mesh,
        compiler_params=pltpu.CompilerParams(needs_layout_passes=False),
        scratch_types=[
            pltpu.VMEM((256,), jnp.int32),
            pltpu.VMEM((16,), jnp.int32),
            pltpu.VMEM((16, F), jnp.float32),
            pltpu.SemaphoreType.DMA,
        ],
    )
    he, = f(idx.reshape(-1), emb_rows)
    return he


def _topk_gather_placeholder(weights, sel0, emb):
    _, idx = lax.top_k(weights, K)
    he = jnp.take_along_axis(
        emb.reshape(B, F, V), idx[:, None, :], axis=2)
    he = jnp.transpose(he, (0, 2, 1)).reshape(B * K, F)
    selk = jnp.take_along_axis(sel0, idx, axis=1).reshape(B * K, 1)
    selk = jnp.broadcast_to(selk, (B * K, 16))
    return he, selk


def kernel(proba, y, embeddings, average_representations):
    weights, sel0 = _tc1a(proba, y)
    idx, selk = _sc2(weights, sel0)
    sums, counts, emb_t = _tc1b(y, embeddings)
    he = _scg(idx, emb_t.reshape(B * V, F))
    return _tc3(he, selk, sums, counts, average_representations)
